# baseline (speedup 1.0000x reference)
import functools

import jax
import jax.numpy as jnp
from jax import lax
from jax.experimental import pallas as pl
from jax.experimental.pallas import tpu as pltpu
from jax.experimental.pallas import tpu_sc as plsc

MAXLEN = 200
EMBED_DIM = 128
BATCH = 1024

NC = 2
NS = 16
LANES = 16
NW = NC * NS
SEQ_PER_W = BATCH // NW
IDX_CHUNK = 100
N_CHUNK = MAXLEN // IDX_CHUNK
NBUF = 4
MAIN_TRIPS = SEQ_PER_W // NBUF


def _body(x_hbm, tok_hbm, pos_a_hbm, pos_b_hbm, out_hbm,
          idx_v, pos_sh, pos_vt, *bufsem):
    bufs = bufsem[:NBUF]
    gsems = bufsem[NBUF:2 * NBUF]
    wsems = bufsem[2 * NBUF:3 * NBUF]
    psems = bufsem[3 * NBUF:4 * NBUF]
    wid = lax.axis_index("s") * NC + lax.axis_index("c")

    pltpu.sync_copy(x_hbm.at[wid], idx_v)
    pltpu.sync_copy(pos_b_hbm, pos_vt)

    @pl.when(lax.axis_index("s") == 0)
    def _():
        pltpu.sync_copy(pos_a_hbm, pos_sh)

    plsc.subcore_barrier()

    def issue_prefill(k):
        pltpu.async_copy(pos_sh, bufs[k].at[pl.ds(0, IDX_CHUNK)], psems[k])

    def wait_prefill(k):
        pltpu.make_async_copy(
            pos_sh, bufs[k].at[pl.ds(0, IDX_CHUNK)], psems[k]).wait()

    def issue_gather(s, k):
        for j in range(N_CHUNK):
            pltpu.async_copy(
                tok_hbm.at[idx_v.at[s * N_CHUNK + j]],
                bufs[k].at[pl.ds(j * IDX_CHUNK, IDX_CHUNK)],
                gsems[k],
                add=(j == 0),
            )

    def wait_gather(k):
        pltpu.make_async_copy(tok_hbm.at[pl.ds(0, MAXLEN)], bufs[k], gsems[k]).wait()

    def add_pos_back(k):
        buf = bufs[k]
        def row_step(r, c2):
            for cc in range(EMBED_DIM // LANES):
                sl = pl.ds(cc * LANES, LANES)
                plsc.addupdate(buf.at[IDX_CHUNK + r, sl], pos_vt[r, sl])
            return c2
        lax.fori_loop(0, IDX_CHUNK, row_step, 0, unroll=4)

    def issue_wb(s, k):
        row = wid * (SEQ_PER_W * MAXLEN) + s * MAXLEN
        pltpu.async_copy(bufs[k], out_hbm.at[pl.ds(row, MAXLEN)], wsems[k])

    def wait_wb(k):
        pltpu.make_async_copy(bufs[k], out_hbm.at[pl.ds(0, MAXLEN)], wsems[k]).wait()

    issue_prefill(0)
    wait_prefill(0)
    issue_gather(0, 0)
    issue_prefill(1)

    def step(u, carry):
        for k in range(NBUF):
            s = u * NBUF + k
            kn = (k + 1) % NBUF
            kp = (k + 2) % NBUF
            wait_gather(k)

            if k == NBUF - 1:
                @pl.when(u < MAIN_TRIPS - 1)
                def _(s=s, kn=kn):
                    wait_prefill(kn)
                    issue_gather(s + 1, kn)
            else:
                wait_prefill(kn)
                issue_gather(s + 1, kn)

            add_pos_back(k)
            issue_wb(s, k)

            if k < 2:
                @pl.when(u > 0)
                def _(kp=kp):
                    wait_wb(kp)
            else:
                wait_wb(kp)
            if k >= 2:
                @pl.when(u < MAIN_TRIPS - 1)
                def _(kp=kp):
                    issue_prefill(kp)
            else:
                issue_prefill(kp)
        return carry

    lax.fori_loop(0, MAIN_TRIPS, step, 0, unroll=False)

    wait_wb(2)
    wait_wb(3)


@jax.jit
def _embed(x2, token_table, pos_a, pos_b):
    mesh = plsc.VectorSubcoreMesh(
        core_axis_name="c", subcore_axis_name="s", num_cores=NC, num_subcores=NS
    )
    run = functools.partial(
        pl.kernel,
        mesh=mesh,
        out_type=jax.ShapeDtypeStruct((BATCH * MAXLEN, EMBED_DIM), jnp.float32),
        scratch_types=[
            pltpu.VMEM((SEQ_PER_W * N_CHUNK, IDX_CHUNK), jnp.int32),
            pltpu.VMEM_SHARED((IDX_CHUNK, EMBED_DIM), jnp.float32),
            pltpu.VMEM((IDX_CHUNK, EMBED_DIM), jnp.float32),
        ]
        + [pltpu.VMEM((MAXLEN, EMBED_DIM), jnp.float32) for _ in range(NBUF)]
        + [pltpu.SemaphoreType.DMA for _ in range(3 * NBUF)],
    )(_body)
    return run(x2, token_table, pos_a, pos_b)


def kernel(x, token_table, pos_table):
    x2 = x.astype(jnp.int32).reshape(NW, SEQ_PER_W * N_CHUNK, IDX_CHUNK)
    pos_a = pos_table[:IDX_CHUNK]
    pos_b = pos_table[IDX_CHUNK:]
    out = _embed(x2, token_table, pos_a, pos_b)
    return out.reshape(BATCH, MAXLEN, EMBED_DIM)
